# Initial kernel scaffold; baseline (speedup 1.0000x reference)
import jax
import jax.numpy as jnp
from jax.experimental import pallas as pl


def kernel(x, W_in, b_in, tod_table, dow_table, node_emb, adaptive_emb):
    raise NotImplementedError("write your pallas kernel here")


# speedup vs baseline: 1.9836x; 1.9836x over previous
import functools

import jax
import jax.numpy as jnp
from jax.experimental import pallas as pl

_B, _T, _N = 16, 12, 1024
_IN_DIM = 3
_STEPS_PER_DAY = 288


def _body(x_ref, w_ref, b_ref, tod_ref, dow_ref, node_ref, adp_ref, out_ref):
    xb = x_ref[0, 0]
    x0 = xb[:, 0:1]
    x1 = xb[:, 1:2]
    x2 = xb[:, 2:3]

    w = w_ref[...]
    bias = b_ref[...]
    xin = x0 * w[0:1, :] + x1 * w[1:2, :] + x2 * w[2:3, :] + bias

    ti = (x1 * float(_STEPS_PER_DAY)).astype(jnp.int32)
    cols = jax.lax.broadcasted_iota(jnp.int32, (_N, _STEPS_PER_DAY), 1)
    oh = (ti == cols).astype(jnp.float32)
    tod_e = jax.lax.dot_general(
        oh, tod_ref[...], (((1,), (0,)), ((), ())),
        preferred_element_type=jnp.float32)

    di = (x2 * 7.0).astype(jnp.int32)
    dow_t = dow_ref[...]
    dow_e = jnp.zeros((_N, 64), jnp.float32)
    for d in range(7):
        dow_e = jnp.where(di == d, dow_t[d:d + 1, :], dow_e)

    out_ref[0, 0] = jnp.concatenate(
        [xin, tod_e, dow_e, node_ref[...], adp_ref[0]], axis=-1)


@jax.jit
def kernel(x, W_in, b_in, tod_table, dow_table, node_emb, adaptive_emb):
    B, T, N, _ = x.shape
    grid = (T, B)
    out = pl.pallas_call(
        _body,
        grid=grid,
        in_specs=[
            pl.BlockSpec((1, 1, N, _IN_DIM), lambda t, b: (b, t, 0, 0)),
            pl.BlockSpec((_IN_DIM, 128), lambda t, b: (0, 0)),
            pl.BlockSpec((1, 128), lambda t, b: (0, 0)),
            pl.BlockSpec((_STEPS_PER_DAY, 64), lambda t, b: (0, 0)),
            pl.BlockSpec((7, 64), lambda t, b: (0, 0)),
            pl.BlockSpec((N, 64), lambda t, b: (0, 0)),
            pl.BlockSpec((1, N, 128), lambda t, b: (t, 0, 0)),
        ],
        out_specs=pl.BlockSpec((1, 1, N, 448), lambda t, b: (b, t, 0, 0)),
        out_shape=jax.ShapeDtypeStruct((B, T, N, 448), jnp.float32),
    )(x, W_in, b_in.reshape(1, 128), tod_table, dow_table, node_emb,
      adaptive_emb)
    return out
